# baseline (speedup 1.0000x reference)
import jax
import jax.numpy as jnp
import numpy as np
from jax.experimental import pallas as pl

_N = 10000
_E = 320000
_D = 128
_DEPTH = 3
_RATE = 0.5


def _ks():
    ks = []
    n = _N
    for _ in range(_DEPTH):
        n = int(np.ceil(_RATE * n))
        ks.append(n)
    return ks


_KS = _ks()


def _gcn_conv(x, edge_index, edge_weight, W, b, num_nodes):
    src = jnp.clip(edge_index[0], 0, num_nodes - 1)
    dst = jnp.clip(edge_index[1], 0, num_nodes - 1)
    loop = jnp.arange(num_nodes, dtype=src.dtype)
    s = jnp.concatenate([src, loop])
    d = jnp.concatenate([dst, loop])
    w = jnp.concatenate([edge_weight, 2.0 * jnp.ones((num_nodes,), dtype=jnp.float32)])
    deg = jax.ops.segment_sum(w, d, num_segments=num_nodes)
    dinv = jnp.where(deg > 0, 1.0 / jnp.sqrt(deg), 0.0)
    norm = dinv[s] * w * dinv[d]
    h = x @ W
    msg = h[s] * norm[:, None]
    if num_nodes == _N and W.shape == (_D, _D):
        out = jax.ops.segment_sum(msg[::-1], d[::-1], num_segments=num_nodes)
    else:
        out = jax.ops.segment_sum(msg, d, num_segments=num_nodes)
    return out + b


def _topk_pool(x, edge_index, edge_weight, p, k):
    n = x.shape[0]
    score = jnp.tanh((x @ p) / jnp.linalg.norm(p))
    perm = jnp.argsort(-score)[:k]
    x_new = x[perm] * score[perm][:, None]
    new_idx = jnp.full((n,), k, dtype=jnp.int32).at[perm].set(jnp.arange(k, dtype=jnp.int32))
    ns = new_idx[edge_index[0]]
    nd = new_idx[edge_index[1]]
    valid = (ns < k) & (nd < k)
    w = edge_weight * valid.astype(jnp.float32)
    ei = jnp.stack([jnp.minimum(ns, k - 1), jnp.minimum(nd, k - 1)])
    return x_new, ei, w, perm


def kernel(x, edge_index, batch, W_down, b_down, p_pool, W_up, b_up):
    ew = (edge_index[0] != edge_index[1]).astype(jnp.float32)
    xc = jax.nn.relu(_gcn_conv(x, edge_index, ew, W_down[0], b_down[0], _N))
    xs = [xc]; eis = [edge_index]; ews = [ew]; sizes = [_N]; perms = []
    ei = edge_index; w = ew; b = batch
    for i in range(_DEPTH):
        k = _KS[i]
        xc, ei, w, perm = _topk_pool(xc, ei, w, p_pool[i], k)
        b = b[perm]
        xc = jax.nn.relu(_gcn_conv(xc, ei, w, W_down[i + 1], b_down[i + 1], k))
        if i < _DEPTH - 1:
            xs.append(xc); eis.append(ei); ews.append(w); sizes.append(k)
        perms.append(perm)
    latent_x = xc; latent_edge = ei; latent_batch = b
    for i in range(_DEPTH):
        j = _DEPTH - 1 - i
        res = xs[j]
        up = jnp.zeros_like(res).at[perms[j]].set(xc)
        xc = res + up
        xc = _gcn_conv(xc, eis[j], ews[j], W_up[i], b_up[i], sizes[j])
        if i < _DEPTH - 1:
            xc = jax.nn.relu(xc)
    return xc, latent_x, latent_edge, latent_batch


# speedup vs baseline: 1.3813x; 1.3813x over previous
import functools

import jax
import jax.numpy as jnp
import numpy as np
from jax import lax
from jax.experimental import pallas as pl
from jax.experimental.pallas import tpu as pltpu
from jax.experimental.pallas import tpu_sc as plsc

_N, _E, _D = 10000, 320000, 128
_DEPTH = 3
_KS = [5000, 2500, 1250]
_NPAD = {10000: 10240, 5000: 5120, 2500: 2560, 1250: 1280}
_NW = 32
_EPT = _E // _NW
_CE = 2000
_SUB = 80

_f32 = jnp.float32
_i32 = jnp.int32


def _mesh():
    return plsc.VectorSubcoreMesh(core_axis_name="c", subcore_axis_name="s")


def _zero_fill(ref, nelem):
    def z16(i, _):
        ref[pl.ds(i * 16, 16)] = jnp.zeros((16,), ref.dtype)
        return 0
    lax.fori_loop(0, nelem // 16, z16, 0)


def _zero_fill_2d(ref, rows, cols):
    def z(i, _):
        r = i // (cols // 16)
        q = i % (cols // 16)
        ref[r, pl.ds(q * 16, 16)] = jnp.zeros((16,), ref.dtype)
        return 0
    lax.fori_loop(0, rows * (cols // 16), z, 0)


def _copy80(dst, src, src_off):
    def c16(i, _):
        dst[pl.ds(i * 16, 16)] = src[pl.ds(src_off + i * 16, 16)]
        return 0
    lax.fori_loop(0, _SUB // 16, c16, 0)


@functools.lru_cache(maxsize=None)
def _edge_init_kernel():
    NP0 = _NPAD[_N]
    nz = NP0 // 16

    @functools.partial(
        pl.kernel,
        out_type=(jax.ShapeDtypeStruct((_E,), _i32),
                  jax.ShapeDtypeStruct((2 * NP0,), _f32)),
        mesh=_mesh(),
        scratch_types=[
            pltpu.VMEM((_CE,), _i32),
            pltpu.VMEM((_CE,), _i32),
            pltpu.VMEM((_CE,), _i32),
            pltpu.VMEM((_SUB,), _i32),
            pltpu.VMEM((_SUB,), _f32),
            pltpu.VMEM((nz,), _f32),
            pltpu.VMEM_SHARED((NP0,), _f32),
        ],
    )
    def k(s_hbm, d_hbm, deff_hbm, deg_hbm, sv, dv, dev, idx80, ones80, zv, degsp):
        c = lax.axis_index("c")
        sid = lax.axis_index("s")
        wid = sid * 2 + c

        def fill16(i, _):
            ones80[pl.ds(i * 16, 16)] = jnp.full((16,), 1.0, _f32)
            return 0
        lax.fori_loop(0, _SUB // 16, fill16, 0)
        _zero_fill(zv, nz)
        pltpu.sync_copy(zv, degsp.at[pl.ds(sid * nz, nz)])
        plsc.subcore_barrier()

        for ch in range(_EPT // _CE):
            base = wid * _EPT + ch * _CE
            pltpu.sync_copy(s_hbm.at[pl.ds(base, _CE)], sv)
            pltpu.sync_copy(d_hbm.at[pl.ds(base, _CE)], dv)

            def b16(i, _):
                s16 = sv[pl.ds(i * 16, 16)]
                d16 = dv[pl.ds(i * 16, 16)]
                io = lax.iota(_i32, 16)
                dev[pl.ds(i * 16, 16)] = jnp.where(s16 != d16, d16, _N + io)
                return 0
            lax.fori_loop(0, _CE // 16, b16, 0)
            pltpu.sync_copy(dev, deff_hbm.at[pl.ds(base, _CE)])
            for sc in range(_CE // _SUB):
                _copy80(idx80, dev, sc * _SUB)
                pltpu.sync_copy(ones80, degsp.at[idx80], add=True)
        plsc.subcore_barrier()
        pltpu.sync_copy(degsp.at[pl.ds(sid * nz, nz)], zv)
        pltpu.sync_copy(zv, deg_hbm.at[pl.ds(c * NP0 + sid * nz, nz)])

    return k


@functools.lru_cache(maxsize=None)
def _pool_a_kernel(n, kk, npc, kp):
    NPT = npc // _NW
    nzk = kp // 16

    @functools.partial(
        pl.kernel,
        out_type=(jax.ShapeDtypeStruct((kp,), _i32),
                  jax.ShapeDtypeStruct((_E,), _i32),
                  jax.ShapeDtypeStruct((_E,), _i32),
                  jax.ShapeDtypeStruct((_E,), _i32),
                  jax.ShapeDtypeStruct((2 * kp,), _f32)),
        mesh=_mesh(),
        scratch_types=[
            pltpu.VMEM((NPT,), _i32),
            pltpu.VMEM((_CE,), _i32),
            pltpu.VMEM((_CE,), _i32),
            pltpu.VMEM((_CE,), _i32),
            pltpu.VMEM((_CE,), _i32),
            pltpu.VMEM((_CE,), _i32),
            pltpu.VMEM((_CE,), _i32),
            pltpu.VMEM((_SUB,), _i32),
            pltpu.VMEM((_SUB,), _i32),
            pltpu.VMEM((_SUB,), _i32),
            pltpu.VMEM((_SUB,), _i32),
            pltpu.VMEM((_SUB,), _f32),
            pltpu.VMEM((nzk,), _f32),
            pltpu.VMEM((npc,), _i32),
            pltpu.VMEM_SHARED((npc,), _i32),
            pltpu.VMEM_SHARED((kp,), _f32),
            pltpu.SemaphoreType.DMA,
        ],
    )
    def k(rank_hbm, s_hbm, d_hbm, deff_hbm,
          perm_hbm, ns_hbm, nd_hbm, ndeff_hbm, deg_hbm,
          rsl, sv, dv, dev, osv, odv, odev, g_s, g_d,
          idx80, val80, ones80, zv, rtmp, rspm, degsp, sem):
        c = lax.axis_index("c")
        sid = lax.axis_index("s")
        wid = sid * 2 + c

        def fill16(i, _):
            ones80[pl.ds(i * 16, 16)] = jnp.full((16,), 1.0, _f32)
            return 0
        lax.fori_loop(0, _SUB // 16, fill16, 0)
        _zero_fill(zv, nzk)
        pltpu.sync_copy(zv, degsp.at[pl.ds(sid * nzk, nzk)])
        @pl.when(sid == 0)
        def _():
            pltpu.sync_copy(rank_hbm, rtmp)
            pltpu.sync_copy(rtmp, rspm)
        pltpu.sync_copy(rank_hbm.at[pl.ds(wid * NPT, NPT)], rsl)

        for sub in range(NPT // _SUB):
            nbase = wid * NPT + sub * _SUB

            def p16(i, _):
                r16 = rsl[pl.ds(sub * _SUB + i * 16, 16)]
                io = lax.iota(_i32, 16)
                idx80[pl.ds(i * 16, 16)] = jnp.minimum(r16, kp - 1)
                val80[pl.ds(i * 16, 16)] = nbase + i * 16 + io
                return 0
            lax.fori_loop(0, _SUB // 16, p16, 0)
            pltpu.sync_copy(val80, perm_hbm.at[idx80])

        plsc.subcore_barrier()

        for ch in range(_EPT // _CE):
            base = wid * _EPT + ch * _CE
            pltpu.sync_copy(s_hbm.at[pl.ds(base, _CE)], sv)
            pltpu.sync_copy(d_hbm.at[pl.ds(base, _CE)], dv)
            pltpu.sync_copy(deff_hbm.at[pl.ds(base, _CE)], dev)

            for sc in range(_CE // _SUB):
                pltpu.async_copy(rspm.at[sv.at[pl.ds(sc * _SUB, _SUB)]],
                                 g_s, sem).wait()
                pltpu.async_copy(rspm.at[dv.at[pl.ds(sc * _SUB, _SUB)]],
                                 g_d, sem).wait()

                def b16(i, _):
                    de16 = dev[pl.ds(sc * _SUB + i * 16, 16)]
                    nid_s = jnp.minimum(g_s[pl.ds(i * 16, 16)], kk)
                    nid_d = jnp.minimum(g_d[pl.ds(i * 16, 16)], kk)
                    io = lax.iota(_i32, 16)
                    valid = (de16 < n) & (nid_s < kk) & (nid_d < kk)
                    osv[pl.ds(sc * _SUB + i * 16, 16)] = jnp.minimum(nid_s, kk - 1)
                    odv[pl.ds(sc * _SUB + i * 16, 16)] = jnp.minimum(nid_d, kk - 1)
                    de_new = jnp.where(valid, nid_d, kk + io)
                    odev[pl.ds(sc * _SUB + i * 16, 16)] = de_new
                    idx80[pl.ds(i * 16, 16)] = de_new
                    return 0
                lax.fori_loop(0, _SUB // 16, b16, 0)
                pltpu.sync_copy(ones80, degsp.at[idx80], add=True)
            pltpu.sync_copy(osv, ns_hbm.at[pl.ds(base, _CE)])
            pltpu.sync_copy(odv, nd_hbm.at[pl.ds(base, _CE)])
            pltpu.sync_copy(odev, ndeff_hbm.at[pl.ds(base, _CE)])
        plsc.subcore_barrier()
        pltpu.sync_copy(degsp.at[pl.ds(sid * nzk, nzk)], zv)
        pltpu.sync_copy(zv, deg_hbm.at[pl.ds(c * kp + sid * nzk, nzk)])

    return k


@functools.lru_cache(maxsize=None)
def _pool_b_kernel(npc, kp):
    KPT = kp // _NW
    SUBP = min(_SUB, KPT)

    @functools.partial(
        pl.kernel,
        out_type=(jax.ShapeDtypeStruct((kp, _D), _f32),
                  jax.ShapeDtypeStruct((kp,), _f32),
                  jax.ShapeDtypeStruct((kp,), _i32)),
        mesh=_mesh(),
        scratch_types=[
            pltpu.VMEM((KPT,), _i32),
            pltpu.VMEM((SUBP,), _f32),
            pltpu.VMEM((SUBP,), _i32),
            pltpu.VMEM((SUBP, _D), _f32),
            pltpu.SemaphoreType.DMA,
        ],
    )
    def k(perm_hbm, x_hbm, score_hbm, b_hbm,
          xg_hbm, sg_hbm, bg_hbm,
          pv, sg80, bg80, rows, sem):
        c = lax.axis_index("c")
        sid = lax.axis_index("s")
        wid = sid * 2 + c
        base = wid * KPT
        pltpu.sync_copy(perm_hbm.at[pl.ds(base, KPT)], pv)
        for sub in range(KPT // SUBP):
            pidx = pv.at[pl.ds(sub * SUBP, SUBP)]
            pltpu.async_copy(x_hbm.at[pidx], rows, sem).wait()
            pltpu.sync_copy(rows, xg_hbm.at[pl.ds(base + sub * SUBP, SUBP), :])
            pltpu.async_copy(score_hbm.at[pidx], sg80, sem).wait()
            pltpu.sync_copy(sg80, sg_hbm.at[pl.ds(base + sub * SUBP, SUBP)])
            pltpu.async_copy(b_hbm.at[pidx], bg80, sem).wait()
            pltpu.sync_copy(bg80, bg_hbm.at[pl.ds(base + sub * SUBP, SUBP)])

    return k


@functools.lru_cache(maxsize=None)
def _agg_kernel(npc):
    RPT = npc // 16

    @functools.partial(
        pl.kernel,
        out_type=jax.ShapeDtypeStruct((2, npc, _D), _f32),
        mesh=_mesh(),
        scratch_types=[
            pltpu.VMEM((_CE,), _i32),
            pltpu.VMEM((_CE,), _i32),
            pltpu.VMEM((_SUB,), _i32),
            pltpu.VMEM((_SUB, _D), _f32),
            pltpu.VMEM((16, _D), _f32),
            pltpu.VMEM_SHARED((npc, _D), _f32),
            pltpu.SemaphoreType.DMA,
        ],
    )
    def k(hs_hbm, s_hbm, deff_hbm, agg_hbm,
          sv, dev, idx80, rows, zrow, aggsp, sem):
        c = lax.axis_index("c")
        sid = lax.axis_index("s")
        wid = sid * 2 + c
        _zero_fill_2d(zrow, 16, _D)
        for q in range(RPT // 16):
            pltpu.sync_copy(zrow, aggsp.at[pl.ds(sid * RPT + q * 16, 16), :])
        plsc.subcore_barrier()
        for ch in range(_EPT // _CE):
            base = wid * _EPT + ch * _CE
            pltpu.sync_copy(s_hbm.at[pl.ds(base, _CE)], sv)
            pltpu.sync_copy(deff_hbm.at[pl.ds(base, _CE)], dev)
            for sc in range(_CE // _SUB):
                _copy80(idx80, dev, sc * _SUB)
                pltpu.async_copy(hs_hbm.at[sv.at[pl.ds(sc * _SUB, _SUB)]],
                                 rows, sem).wait()
                pltpu.sync_copy(rows, aggsp.at[idx80], add=True)
        plsc.subcore_barrier()
        for q in range(RPT // 16):
            pltpu.sync_copy(aggsp.at[pl.ds(sid * RPT + q * 16, 16), :], zrow)
            pltpu.sync_copy(zrow, agg_hbm.at[c, pl.ds(sid * RPT + q * 16, 16), :])

    return k


@functools.lru_cache(maxsize=None)
def _unpool_kernel(kp, npj):
    KPT = kp // _NW
    SUBP = min(_SUB, KPT)
    RPT = npj // 16

    @functools.partial(
        pl.kernel,
        out_type=jax.ShapeDtypeStruct((2, npj, _D), _f32),
        mesh=_mesh(),
        scratch_types=[
            pltpu.VMEM((KPT,), _i32),
            pltpu.VMEM((SUBP,), _i32),
            pltpu.VMEM((SUBP, _D), _f32),
            pltpu.VMEM((16, _D), _f32),
            pltpu.VMEM_SHARED((npj, _D), _f32),
        ],
    )
    def k(xc_hbm, perm_hbm, up_hbm, pv, idx, rows, zrow, upsp):
        c = lax.axis_index("c")
        sid = lax.axis_index("s")
        wid = sid * 2 + c
        base = wid * KPT
        _zero_fill_2d(zrow, 16, _D)
        for q in range(RPT // 16):
            pltpu.sync_copy(zrow, upsp.at[pl.ds(sid * RPT + q * 16, 16), :])
        plsc.subcore_barrier()
        pltpu.sync_copy(perm_hbm.at[pl.ds(base, KPT)], pv)
        for sub in range(KPT // SUBP):
            def c16(i, _):
                idx[pl.ds(i * 16, 16)] = pv[pl.ds(sub * SUBP + i * 16, 16)]
                return 0
            lax.fori_loop(0, SUBP // 16, c16, 0)
            pltpu.sync_copy(xc_hbm.at[pl.ds(base + sub * SUBP, SUBP), :], rows)
            pltpu.sync_copy(rows, upsp.at[idx], add=True)
        plsc.subcore_barrier()
        for q in range(RPT // 16):
            pltpu.sync_copy(upsp.at[pl.ds(sid * RPT + q * 16, 16), :], zrow)
            pltpu.sync_copy(zrow, up_hbm.at[c, pl.ds(sid * RPT + q * 16, 16), :])

    return k


def _prep(x, sg, degA, degB, W, npc):
    def body(x_ref, sg_ref, dA, dB, W_ref, h_ref, hs_ref, dv_ref):
        xb = x_ref[...] * sg_ref[...]
        h = jnp.dot(xb, W_ref[...], preferred_element_type=_f32)
        dv = 1.0 / jnp.sqrt(dA[...] + dB[...] + 2.0)
        h_ref[...] = h
        hs_ref[...] = h * dv
        dv_ref[...] = dv
    return pl.pallas_call(
        body,
        grid=(npc // 256,),
        in_specs=[pl.BlockSpec((256, _D), lambda i: (i, 0)),
                  pl.BlockSpec((256, 1), lambda i: (i, 0)),
                  pl.BlockSpec((256, 1), lambda i: (i, 0)),
                  pl.BlockSpec((256, 1), lambda i: (i, 0)),
                  pl.BlockSpec((_D, _D), lambda i: (0, 0))],
        out_specs=[pl.BlockSpec((256, _D), lambda i: (i, 0)),
                   pl.BlockSpec((256, _D), lambda i: (i, 0)),
                   pl.BlockSpec((256, 1), lambda i: (i, 0))],
        out_shape=[jax.ShapeDtypeStruct((npc, _D), _f32),
                   jax.ShapeDtypeStruct((npc, _D), _f32),
                   jax.ShapeDtypeStruct((npc, 1), _f32)],
    )(x, sg, degA, degB, W)


def _prep_up(res, uA, uB, degA, degB, W, npc):
    def body(r_ref, a_ref, b2_ref, dA, dB, W_ref, h_ref, hs_ref, dv_ref):
        xb = r_ref[...] + a_ref[...] + b2_ref[...]
        h = jnp.dot(xb, W_ref[...], preferred_element_type=_f32)
        dv = 1.0 / jnp.sqrt(dA[...] + dB[...] + 2.0)
        h_ref[...] = h
        hs_ref[...] = h * dv
        dv_ref[...] = dv
    return pl.pallas_call(
        body,
        grid=(npc // 256,),
        in_specs=[pl.BlockSpec((256, _D), lambda i: (i, 0)),
                  pl.BlockSpec((256, _D), lambda i: (i, 0)),
                  pl.BlockSpec((256, _D), lambda i: (i, 0)),
                  pl.BlockSpec((256, 1), lambda i: (i, 0)),
                  pl.BlockSpec((256, 1), lambda i: (i, 0)),
                  pl.BlockSpec((_D, _D), lambda i: (0, 0))],
        out_specs=[pl.BlockSpec((256, _D), lambda i: (i, 0)),
                   pl.BlockSpec((256, _D), lambda i: (i, 0)),
                   pl.BlockSpec((256, 1), lambda i: (i, 0))],
        out_shape=[jax.ShapeDtypeStruct((npc, _D), _f32),
                   jax.ShapeDtypeStruct((npc, _D), _f32),
                   jax.ShapeDtypeStruct((npc, 1), _f32)],
    )(res, uA, uB, degA, degB, W)


def _epi(aggA, aggB, h, dv, b, n, npc, relu):
    def body(aA, aB, h_ref, dv_ref, b_ref, o_ref):
        i = pl.program_id(0)
        rows = i * 256 + lax.broadcasted_iota(_i32, (256, 1), 0)
        dvv = dv_ref[...]
        val = dvv * (aA[...] + aB[...]) + (2.0 * dvv * dvv) * h_ref[...] + b_ref[...]
        if relu:
            val = jnp.maximum(val, 0.0)
        o_ref[...] = jnp.where(rows < n, val, 0.0)
    return pl.pallas_call(
        body,
        grid=(npc // 256,),
        in_specs=[pl.BlockSpec((256, _D), lambda i: (i, 0)),
                  pl.BlockSpec((256, _D), lambda i: (i, 0)),
                  pl.BlockSpec((256, _D), lambda i: (i, 0)),
                  pl.BlockSpec((256, 1), lambda i: (i, 0)),
                  pl.BlockSpec((1, _D), lambda i: (0, 0))],
        out_specs=pl.BlockSpec((256, _D), lambda i: (i, 0)),
        out_shape=jax.ShapeDtypeStruct((npc, _D), _f32),
    )(aggA, aggB, h, dv, b)


def _score(x, p, npc):
    def body(x_ref, p_ref, o_ref):
        pv = p_ref[...]
        nrm = jnp.sqrt(jnp.sum(pv * pv))
        t = jnp.dot(x_ref[...], pv, preferred_element_type=_f32) / nrm
        o_ref[...] = jnp.tanh(t)
    return pl.pallas_call(
        body,
        grid=(npc // 256,),
        in_specs=[pl.BlockSpec((256, _D), lambda i: (i, 0)),
                  pl.BlockSpec((_D, 1), lambda i: (0, 0))],
        out_specs=pl.BlockSpec((256, 1), lambda i: (i, 0)),
        out_shape=jax.ShapeDtypeStruct((npc, 1), _f32),
    )(x, p)


def _rank(score_col, score_row, n, npc):
    BLK = 512

    def body(sc_ref, sr_ref, o_ref):
        i = pl.program_id(0)
        rid = i * BLK + lax.broadcasted_iota(_i32, (BLK, 1), 0)
        key_i = jnp.where(rid < n, sc_ref[...], -2.0)
        cnt = jnp.zeros((BLK, 1), _f32)
        for j in range(npc // BLK):
            sj = sr_ref[:, pl.ds(j * BLK, BLK)]
            cid = j * BLK + lax.broadcasted_iota(_i32, (1, BLK), 1)
            key_j = jnp.where(cid < n, sj, -2.0)
            gt = key_j > key_i
            tie = (key_j == key_i) & (cid < rid)
            cnt = cnt + jnp.sum(jnp.where(gt | tie, 1.0, 0.0),
                                axis=1, keepdims=True)
        o_ref[...] = cnt.astype(_i32)
    return pl.pallas_call(
        body,
        grid=(npc // BLK,),
        in_specs=[pl.BlockSpec((BLK, 1), lambda i: (i, 0)),
                  pl.BlockSpec((1, npc), lambda i: (0, 0))],
        out_specs=pl.BlockSpec((BLK, 1), lambda i: (i, 0)),
        out_shape=jax.ShapeDtypeStruct((npc, 1), _i32),
    )(score_col, score_row)


def kernel(x, edge_index, batch, W_down, b_down, p_pool, W_up, b_up):
    NP0 = _NPAD[_N]
    x_p = jnp.zeros((NP0, _D), _f32).at[:_N].set(x)
    b_p = jnp.zeros((NP0,), _i32).at[:_N].set(batch)
    ones0 = jnp.ones((NP0, 1), _f32)

    s0 = edge_index[0]
    d0 = edge_index[1]
    deff0, deg0 = _edge_init_kernel()(s0, d0)

    h, hs, dv = _prep(x_p, ones0, deg0[:NP0][:, None], deg0[NP0:][:, None],
                      W_down[0], NP0)
    agg = _agg_kernel(NP0)(hs, s0, deff0)
    xc = _epi(agg[0], agg[1], h, dv, b_down[0][None, :], _N, NP0, True)

    xs = [xc]
    saved = [(s0, deff0, deg0, _N, NP0)]
    perms = []
    b_cur = b_p
    s_arr, d_arr, deff = s0, d0, deff0
    n_cur, np_cur = _N, NP0
    for i in range(_DEPTH):
        kk = _KS[i]
        kp = _NPAD[kk]
        score = _score(xc, p_pool[i][:, None], np_cur)
        rank = _rank(score, score.reshape(1, np_cur), n_cur, np_cur)
        perm, new_s, new_d, new_deff, degp = _pool_a_kernel(
            n_cur, kk, np_cur, kp)(rank[:, 0], s_arr, d_arr, deff)
        xg, sg, bg = _pool_b_kernel(np_cur, kp)(
            perm, xc, score[:, 0], b_cur)
        h, hs, dv = _prep(xg, sg[:, None], degp[:kp][:, None], degp[kp:][:, None],
                          W_down[i + 1], kp)
        agg = _agg_kernel(kp)(hs, new_s, new_deff)
        xc = _epi(agg[0], agg[1], h, dv, b_down[i + 1][None, :], kk, kp, True)
        perms.append(perm)
        b_cur = bg
        s_arr, d_arr, deff = new_s, new_d, new_deff
        n_cur, np_cur = kk, kp
        if i < _DEPTH - 1:
            xs.append(xc)
            saved.append((new_s, new_deff, degp, kk, kp))

    latent_x = xc[:_KS[-1]]
    latent_edge = jnp.stack([s_arr, d_arr])
    latent_batch = b_cur[:_KS[-1]]

    for i in range(_DEPTH):
        j = _DEPTH - 1 - i
        s_j, deff_j, deg_j, n_j, np_j = saved[j]
        up = _unpool_kernel(np_cur, np_j)(xc, perms[j])
        h, hs, dv = _prep_up(xs[j], up[0], up[1],
                             deg_j[:np_j][:, None], deg_j[np_j:][:, None],
                             W_up[i], np_j)
        agg = _agg_kernel(np_j)(hs, s_j, deff_j)
        xc = _epi(agg[0], agg[1], h, dv, b_up[i][None, :],
                  n_j, np_j, i < _DEPTH - 1)
        n_cur, np_cur = n_j, np_j

    return xc[:_N], latent_x, latent_edge, latent_batch


# speedup vs baseline: 1.3890x; 1.0055x over previous
import functools

import jax
import jax.numpy as jnp
import numpy as np
from jax import lax
from jax.experimental import pallas as pl
from jax.experimental.pallas import tpu as pltpu
from jax.experimental.pallas import tpu_sc as plsc

_N, _E, _D = 10000, 320000, 128
_DEPTH = 3
_KS = [5000, 2500, 1250]
_NPAD = {10000: 10240, 5000: 5120, 2500: 2560, 1250: 1280}
_NW = 32
_EPT = _E // _NW
_CE = 2000
_SUB = 80

_f32 = jnp.float32
_i32 = jnp.int32


def _mesh():
    return plsc.VectorSubcoreMesh(core_axis_name="c", subcore_axis_name="s")


def _zero_fill(ref, nelem):
    def z16(i, _):
        ref[pl.ds(i * 16, 16)] = jnp.zeros((16,), ref.dtype)
        return 0
    lax.fori_loop(0, nelem // 16, z16, 0)


def _ones_fill(ref, nelem):
    def o16(i, _):
        ref[pl.ds(i * 16, 16)] = jnp.full((16,), 1.0, _f32)
        return 0
    lax.fori_loop(0, nelem // 16, o16, 0)


def _zero_fill_2d(ref, rows, cols):
    def z(i, _):
        r = i // (cols // 16)
        q = i % (cols // 16)
        ref[r, pl.ds(q * 16, 16)] = jnp.zeros((16,), ref.dtype)
        return 0
    lax.fori_loop(0, rows * (cols // 16), z, 0)


def _copy80(dst, src, src_off):
    def c16(i, _):
        dst[pl.ds(i * 16, 16)] = src[pl.ds(src_off + i * 16, 16)]
        return 0
    lax.fori_loop(0, _SUB // 16, c16, 0)


@functools.lru_cache(maxsize=None)
def _edge_init_kernel():
    NP0 = _NPAD[_N]
    nz = NP0 // 16

    @functools.partial(
        pl.kernel,
        out_type=(jax.ShapeDtypeStruct((_E,), _i32),
                  jax.ShapeDtypeStruct((2 * NP0,), _f32)),
        mesh=_mesh(),
        scratch_types=[
            pltpu.VMEM((_CE,), _i32),
            pltpu.VMEM((_CE,), _i32),
            pltpu.VMEM((_CE,), _i32),
            pltpu.VMEM((_SUB,), _i32),
            pltpu.VMEM((_SUB,), _i32),
            pltpu.VMEM((_SUB,), _f32),
            pltpu.VMEM((nz,), _f32),
            pltpu.VMEM_SHARED((NP0,), _f32),
            pltpu.SemaphoreType.DMA,
            pltpu.SemaphoreType.DMA,
        ],
    )
    def k(s_hbm, d_hbm, deff_hbm, deg_hbm,
          sv, dv, dev, di0, di1, ones_v, zv, degsp, sem0, sem1):
        c = lax.axis_index("c")
        sid = lax.axis_index("s")
        wid = sid * 2 + c
        di = [di0, di1]
        sems = [sem0, sem1]

        _ones_fill(ones_v, _SUB)
        _zero_fill(zv, nz)
        pltpu.sync_copy(zv, degsp.at[pl.ds(sid * nz, nz)])
        plsc.subcore_barrier()

        t = 0
        for ch in range(_EPT // _CE):
            base = wid * _EPT + ch * _CE
            pltpu.sync_copy(s_hbm.at[pl.ds(base, _CE)], sv)
            pltpu.sync_copy(d_hbm.at[pl.ds(base, _CE)], dv)

            for sc in range(_CE // _SUB):
                b = t % 2
                if t >= 2:
                    pltpu.make_async_copy(ones_v, degsp.at[di[b]], sems[b]).wait()

                def b16(i, _):
                    off = sc * _SUB + i * 16
                    s16 = sv[pl.ds(off, 16)]
                    d16 = dv[pl.ds(off, 16)]
                    io = lax.iota(_i32, 16)
                    de = jnp.where(s16 != d16, d16, _N + io)
                    dev[pl.ds(off, 16)] = de
                    di[b][pl.ds(i * 16, 16)] = de
                    return 0
                lax.fori_loop(0, _SUB // 16, b16, 0)
                pltpu.async_copy(ones_v, degsp.at[di[b]], sems[b], add=True)
                t += 1
            pltpu.sync_copy(dev, deff_hbm.at[pl.ds(base, _CE)])
        pltpu.make_async_copy(ones_v, degsp.at[di0], sem0).wait()
        pltpu.make_async_copy(ones_v, degsp.at[di1], sem1).wait()
        plsc.subcore_barrier()
        pltpu.sync_copy(degsp.at[pl.ds(sid * nz, nz)], zv)
        pltpu.sync_copy(zv, deg_hbm.at[pl.ds(c * NP0 + sid * nz, nz)])

    return k


@functools.lru_cache(maxsize=None)
def _pool_a_kernel(n, kk, npc, kp):
    NPT = npc // _NW
    nzk = kp // 16

    @functools.partial(
        pl.kernel,
        out_type=(jax.ShapeDtypeStruct((kp,), _i32),
                  jax.ShapeDtypeStruct((_E,), _i32),
                  jax.ShapeDtypeStruct((_E,), _i32),
                  jax.ShapeDtypeStruct((_E,), _i32),
                  jax.ShapeDtypeStruct((2 * kp,), _f32)),
        mesh=_mesh(),
        scratch_types=[
            pltpu.VMEM((NPT,), _i32),
            pltpu.VMEM((_SUB,), _i32),
            pltpu.VMEM((_SUB,), _i32),
            pltpu.VMEM((_CE,), _i32),
            pltpu.VMEM((_CE,), _i32),
            pltpu.VMEM((_CE,), _i32),
            pltpu.VMEM((_CE,), _i32),
            pltpu.VMEM((_CE,), _i32),
            pltpu.VMEM((_CE,), _i32),
            pltpu.VMEM((_SUB,), _i32),
            pltpu.VMEM((_SUB,), _i32),
            pltpu.VMEM((_SUB,), _i32),
            pltpu.VMEM((_SUB,), _i32),
            pltpu.VMEM((_SUB,), _f32),
            pltpu.VMEM((nzk,), _f32),
            pltpu.VMEM((npc,), _i32),
            pltpu.VMEM_SHARED((npc,), _i32),
            pltpu.VMEM_SHARED((kp,), _f32),
            pltpu.SemaphoreType.DMA,
            pltpu.SemaphoreType.DMA,
            pltpu.SemaphoreType.DMA,
        ],
    )
    def k(rank_hbm, s_hbm, d_hbm, deff_hbm,
          perm_hbm, ns_hbm, nd_hbm, ndeff_hbm, deg_hbm,
          rsl, pidx, pval, sv, dv, dev, osv, odv, odev, g_s, g_d,
          di0, di1, ones_v, zv, rtmp, rspm, degsp, sem0, sem1, sem2):
        c = lax.axis_index("c")
        sid = lax.axis_index("s")
        wid = sid * 2 + c
        di = [di0, di1]
        sems = [sem0, sem1]

        _ones_fill(ones_v, _SUB)
        _zero_fill(zv, nzk)
        pltpu.sync_copy(zv, degsp.at[pl.ds(sid * nzk, nzk)])
        @pl.when(sid == 0)
        def _():
            pltpu.sync_copy(rank_hbm, rtmp)
            pltpu.sync_copy(rtmp, rspm)
        pltpu.sync_copy(rank_hbm.at[pl.ds(wid * NPT, NPT)], rsl)

        for sub in range(NPT // _SUB):
            def p16(i, _):
                r16 = rsl[pl.ds(sub * _SUB + i * 16, 16)]
                io = lax.iota(_i32, 16)
                pidx[pl.ds(i * 16, 16)] = jnp.minimum(r16, kp - 1)
                pval[pl.ds(i * 16, 16)] = wid * NPT + sub * _SUB + i * 16 + io
                return 0
            lax.fori_loop(0, _SUB // 16, p16, 0)
            pltpu.sync_copy(pval, perm_hbm.at[pidx])

        plsc.subcore_barrier()

        t = 0
        for ch in range(_EPT // _CE):
            base = wid * _EPT + ch * _CE
            pltpu.sync_copy(s_hbm.at[pl.ds(base, _CE)], sv)
            pltpu.sync_copy(d_hbm.at[pl.ds(base, _CE)], dv)
            pltpu.sync_copy(deff_hbm.at[pl.ds(base, _CE)], dev)

            for sc in range(_CE // _SUB):
                gs = pltpu.async_copy(
                    rspm.at[sv.at[pl.ds(sc * _SUB, _SUB)]], g_s, sem2)
                gd = pltpu.async_copy(
                    rspm.at[dv.at[pl.ds(sc * _SUB, _SUB)]], g_d, sem2)
                gs.wait()
                gd.wait()
                b = t % 2
                if t >= 2:
                    pltpu.make_async_copy(ones_v, degsp.at[di[b]], sems[b]).wait()

                def b16(i, _):
                    off = sc * _SUB + i * 16
                    de16 = dev[pl.ds(off, 16)]
                    nid_s = jnp.minimum(g_s[pl.ds(i * 16, 16)], kk)
                    nid_d = jnp.minimum(g_d[pl.ds(i * 16, 16)], kk)
                    io = lax.iota(_i32, 16)
                    valid = (de16 < n) & (nid_s < kk) & (nid_d < kk)
                    osv[pl.ds(off, 16)] = jnp.minimum(nid_s, kk - 1)
                    odv[pl.ds(off, 16)] = jnp.minimum(nid_d, kk - 1)
                    de_new = jnp.where(valid, nid_d, kk + io)
                    odev[pl.ds(off, 16)] = de_new
                    di[b][pl.ds(i * 16, 16)] = de_new
                    return 0
                lax.fori_loop(0, _SUB // 16, b16, 0)
                pltpu.async_copy(ones_v, degsp.at[di[b]], sems[b], add=True)
                t += 1
            pltpu.sync_copy(osv, ns_hbm.at[pl.ds(base, _CE)])
            pltpu.sync_copy(odv, nd_hbm.at[pl.ds(base, _CE)])
            pltpu.sync_copy(odev, ndeff_hbm.at[pl.ds(base, _CE)])
        pltpu.make_async_copy(ones_v, degsp.at[di0], sem0).wait()
        pltpu.make_async_copy(ones_v, degsp.at[di1], sem1).wait()
        plsc.subcore_barrier()
        pltpu.sync_copy(degsp.at[pl.ds(sid * nzk, nzk)], zv)
        pltpu.sync_copy(zv, deg_hbm.at[pl.ds(c * kp + sid * nzk, nzk)])

    return k


@functools.lru_cache(maxsize=None)
def _pool_b_kernel(npc, kp):
    KPT = kp // _NW

    SUBP = min(_SUB, KPT)

    @functools.partial(
        pl.kernel,
        out_type=(jax.ShapeDtypeStruct((kp, _D), _f32),
                  jax.ShapeDtypeStruct((kp,), _f32),
                  jax.ShapeDtypeStruct((kp,), _i32)),
        mesh=_mesh(),
        scratch_types=[
            pltpu.VMEM((KPT,), _i32),
            pltpu.VMEM((SUBP,), _f32),
            pltpu.VMEM((SUBP,), _i32),
            pltpu.VMEM((SUBP, _D), _f32),
            pltpu.SemaphoreType.DMA,
            pltpu.SemaphoreType.DMA,
            pltpu.SemaphoreType.DMA,
        ],
    )
    def k(perm_hbm, x_hbm, score_hbm, b_hbm,
          xg_hbm, sg_hbm, bg_hbm,
          pv, sgv, bgv, rows, sem0, sem1, sem2):
        c = lax.axis_index("c")
        sid = lax.axis_index("s")
        wid = sid * 2 + c
        base = wid * KPT
        pltpu.sync_copy(perm_hbm.at[pl.ds(base, KPT)], pv)
        for sub in range(KPT // SUBP):
            pidx = pv.at[pl.ds(sub * SUBP, SUBP)]
            cx = pltpu.async_copy(x_hbm.at[pidx], rows, sem0)
            cs = pltpu.async_copy(score_hbm.at[pidx], sgv, sem1)
            cb = pltpu.async_copy(b_hbm.at[pidx], bgv, sem2)
            cx.wait()
            cs.wait()
            cb.wait()
            o = base + sub * SUBP
            pltpu.sync_copy(rows, xg_hbm.at[pl.ds(o, SUBP), :])
            pltpu.sync_copy(sgv, sg_hbm.at[pl.ds(o, SUBP)])
            pltpu.sync_copy(bgv, bg_hbm.at[pl.ds(o, SUBP)])

    return k


@functools.lru_cache(maxsize=None)
def _agg_kernel(npc):
    RPT = npc // 16
    NSUB = _EPT // _SUB

    @functools.partial(
        pl.kernel,
        out_type=jax.ShapeDtypeStruct((2, npc, _D), _f32),
        mesh=_mesh(),
        scratch_types=[
            pltpu.VMEM((_EPT,), _i32),
            pltpu.VMEM((_EPT,), _i32),
            pltpu.VMEM((_SUB,), _i32),
            pltpu.VMEM((_SUB,), _i32),
            pltpu.VMEM((_SUB, _D), _f32),
            pltpu.VMEM((_SUB, _D), _f32),
            pltpu.VMEM((16, _D), _f32),
            pltpu.VMEM_SHARED((npc, _D), _f32),
            pltpu.SemaphoreType.DMA,
            pltpu.SemaphoreType.DMA,
            pltpu.SemaphoreType.DMA,
            pltpu.SemaphoreType.DMA,
        ],
    )
    def k(hs_hbm, s_hbm, deff_hbm, agg_hbm,
          sv, dev, di0, di1, r0, r1, zrow, aggsp, gs0, gs1, ss0, ss1):
        c = lax.axis_index("c")
        sid = lax.axis_index("s")
        wid = sid * 2 + c
        di = [di0, di1]
        rows = [r0, r1]
        gsem = [gs0, gs1]
        ssem = [ss0, ss1]
        _zero_fill_2d(zrow, 16, _D)
        for q in range(RPT // 16):
            pltpu.sync_copy(zrow, aggsp.at[pl.ds(sid * RPT + q * 16, 16), :])
        plsc.subcore_barrier()

        base0 = wid * _EPT
        pltpu.sync_copy(s_hbm.at[pl.ds(base0, _EPT)], sv)
        pltpu.sync_copy(deff_hbm.at[pl.ds(base0, _EPT)], dev)

        def fill_di(b, t):
            def c16(i, _):
                di[b][pl.ds(i * 16, 16)] = dev[pl.ds(t * _SUB + i * 16, 16)]
                return 0
            lax.fori_loop(0, _SUB // 16, c16, 0)

        def start_gather(b, t):
            pltpu.async_copy(hs_hbm.at[sv.at[pl.ds(t * _SUB, _SUB)]],
                             rows[b], gsem[b])

        fill_di(0, 0)
        start_gather(0, 0)
        for t in range(NSUB):
            b = t % 2
            nb = (t + 1) % 2
            if t + 1 < NSUB:
                if t >= 1:
                    pltpu.make_async_copy(rows[nb], aggsp.at[di[nb]],
                                          ssem[nb]).wait()
                fill_di(nb, t + 1)
                start_gather(nb, t + 1)
            pltpu.make_async_copy(hs_hbm.at[sv.at[pl.ds(t * _SUB, _SUB)]],
                                  rows[b], gsem[b]).wait()
            pltpu.async_copy(rows[b], aggsp.at[di[b]], ssem[b], add=True)
        pltpu.make_async_copy(rows[0], aggsp.at[di[0]], ssem[0]).wait()
        pltpu.make_async_copy(rows[1], aggsp.at[di[1]], ssem[1]).wait()
        plsc.subcore_barrier()
        half = min(_SUB, RPT)
        for q in range(RPT // half):
            pltpu.sync_copy(
                aggsp.at[pl.ds(sid * RPT + q * half, half), :],
                r0.at[pl.ds(0, half), :])
            pltpu.sync_copy(
                r0.at[pl.ds(0, half), :],
                agg_hbm.at[c, pl.ds(sid * RPT + q * half, half), :])

    return k


@functools.lru_cache(maxsize=None)
def _unpool_kernel(kp, npj):
    KPT = kp // _NW
    RPT = npj // 16

    SUBP = min(_SUB, KPT)

    @functools.partial(
        pl.kernel,
        out_type=jax.ShapeDtypeStruct((2, npj, _D), _f32),
        mesh=_mesh(),
        scratch_types=[
            pltpu.VMEM((KPT,), _i32),
            pltpu.VMEM((SUBP,), _i32),
            pltpu.VMEM((SUBP, _D), _f32),
            pltpu.VMEM((16, _D), _f32),
            pltpu.VMEM_SHARED((npj, _D), _f32),
        ],
    )
    def k(xc_hbm, perm_hbm, up_hbm, pv, idx80, rows, zrow, upsp):
        c = lax.axis_index("c")
        sid = lax.axis_index("s")
        wid = sid * 2 + c
        base = wid * KPT
        _zero_fill_2d(zrow, 16, _D)
        for q in range(RPT // 16):
            pltpu.sync_copy(zrow, upsp.at[pl.ds(sid * RPT + q * 16, 16), :])
        plsc.subcore_barrier()
        pltpu.sync_copy(perm_hbm.at[pl.ds(base, KPT)], pv)
        for sub in range(KPT // SUBP):
            def c16(i, _):
                idx80[pl.ds(i * 16, 16)] = pv[pl.ds(sub * SUBP + i * 16, 16)]
                return 0
            lax.fori_loop(0, SUBP // 16, c16, 0)
            pltpu.sync_copy(xc_hbm.at[pl.ds(base + sub * SUBP, SUBP), :], rows)
            pltpu.sync_copy(rows, upsp.at[idx80], add=True)
        plsc.subcore_barrier()
        hh = min(SUBP, RPT)
        for q in range(RPT // hh):
            pltpu.sync_copy(upsp.at[pl.ds(sid * RPT + q * hh, hh), :],
                            rows.at[pl.ds(0, hh), :])
            pltpu.sync_copy(rows.at[pl.ds(0, hh), :],
                            up_hbm.at[c, pl.ds(sid * RPT + q * hh, hh), :])

    return k


def _prep(x, sg, degA, degB, W, npc):
    def body(x_ref, sg_ref, dA, dB, W_ref, h_ref, hs_ref, dv_ref):
        xb = x_ref[...] * sg_ref[...]
        h = jnp.dot(xb, W_ref[...], preferred_element_type=_f32)
        dv = 1.0 / jnp.sqrt(dA[...] + dB[...] + 2.0)
        h_ref[...] = h
        hs_ref[...] = h * dv
        dv_ref[...] = dv
    return pl.pallas_call(
        body,
        grid=(npc // 256,),
        in_specs=[pl.BlockSpec((256, _D), lambda i: (i, 0)),
                  pl.BlockSpec((256, 1), lambda i: (i, 0)),
                  pl.BlockSpec((256, 1), lambda i: (i, 0)),
                  pl.BlockSpec((256, 1), lambda i: (i, 0)),
                  pl.BlockSpec((_D, _D), lambda i: (0, 0))],
        out_specs=[pl.BlockSpec((256, _D), lambda i: (i, 0)),
                   pl.BlockSpec((256, _D), lambda i: (i, 0)),
                   pl.BlockSpec((256, 1), lambda i: (i, 0))],
        out_shape=[jax.ShapeDtypeStruct((npc, _D), _f32),
                   jax.ShapeDtypeStruct((npc, _D), _f32),
                   jax.ShapeDtypeStruct((npc, 1), _f32)],
    )(x, sg, degA, degB, W)


def _prep_up(res, uA, uB, degA, degB, W, npc):
    def body(r_ref, a_ref, b2_ref, dA, dB, W_ref, h_ref, hs_ref, dv_ref):
        xb = r_ref[...] + a_ref[...] + b2_ref[...]
        h = jnp.dot(xb, W_ref[...], preferred_element_type=_f32)
        dv = 1.0 / jnp.sqrt(dA[...] + dB[...] + 2.0)
        h_ref[...] = h
        hs_ref[...] = h * dv
        dv_ref[...] = dv
    return pl.pallas_call(
        body,
        grid=(npc // 256,),
        in_specs=[pl.BlockSpec((256, _D), lambda i: (i, 0)),
                  pl.BlockSpec((256, _D), lambda i: (i, 0)),
                  pl.BlockSpec((256, _D), lambda i: (i, 0)),
                  pl.BlockSpec((256, 1), lambda i: (i, 0)),
                  pl.BlockSpec((256, 1), lambda i: (i, 0)),
                  pl.BlockSpec((_D, _D), lambda i: (0, 0))],
        out_specs=[pl.BlockSpec((256, _D), lambda i: (i, 0)),
                   pl.BlockSpec((256, _D), lambda i: (i, 0)),
                   pl.BlockSpec((256, 1), lambda i: (i, 0))],
        out_shape=[jax.ShapeDtypeStruct((npc, _D), _f32),
                   jax.ShapeDtypeStruct((npc, _D), _f32),
                   jax.ShapeDtypeStruct((npc, 1), _f32)],
    )(res, uA, uB, degA, degB, W)


def _epi(aggA, aggB, h, dv, b, n, npc, relu):
    def body(aA, aB, h_ref, dv_ref, b_ref, o_ref):
        i = pl.program_id(0)
        rows = i * 256 + lax.broadcasted_iota(_i32, (256, 1), 0)
        dvv = dv_ref[...]
        val = dvv * (aA[...] + aB[...]) + (2.0 * dvv * dvv) * h_ref[...] + b_ref[...]
        if relu:
            val = jnp.maximum(val, 0.0)
        o_ref[...] = jnp.where(rows < n, val, 0.0)
    return pl.pallas_call(
        body,
        grid=(npc // 256,),
        in_specs=[pl.BlockSpec((256, _D), lambda i: (i, 0)),
                  pl.BlockSpec((256, _D), lambda i: (i, 0)),
                  pl.BlockSpec((256, _D), lambda i: (i, 0)),
                  pl.BlockSpec((256, 1), lambda i: (i, 0)),
                  pl.BlockSpec((1, _D), lambda i: (0, 0))],
        out_specs=pl.BlockSpec((256, _D), lambda i: (i, 0)),
        out_shape=jax.ShapeDtypeStruct((npc, _D), _f32),
    )(aggA, aggB, h, dv, b)


def _score(x, p, npc):
    def body(x_ref, p_ref, o_ref):
        pv = p_ref[...]
        nrm = jnp.sqrt(jnp.sum(pv * pv))
        t = jnp.dot(x_ref[...], pv, preferred_element_type=_f32) / nrm
        o_ref[...] = jnp.tanh(t)
    return pl.pallas_call(
        body,
        grid=(npc // 256,),
        in_specs=[pl.BlockSpec((256, _D), lambda i: (i, 0)),
                  pl.BlockSpec((_D, 1), lambda i: (0, 0))],
        out_specs=pl.BlockSpec((256, 1), lambda i: (i, 0)),
        out_shape=jax.ShapeDtypeStruct((npc, 1), _f32),
    )(x, p)


def _rank(score_col, score_row, n, npc):
    BLK = 512

    def body(sc_ref, sr_ref, o_ref):
        i = pl.program_id(0)
        rid = i * BLK + lax.broadcasted_iota(_i32, (BLK, 1), 0)
        key_i = jnp.where(rid < n, sc_ref[...], -2.0)
        cnt = jnp.zeros((BLK, 1), _f32)
        for j in range(npc // BLK):
            sj = sr_ref[:, pl.ds(j * BLK, BLK)]
            cid = j * BLK + lax.broadcasted_iota(_i32, (1, BLK), 1)
            key_j = jnp.where(cid < n, sj, -2.0)
            gt = key_j > key_i
            tie = (key_j == key_i) & (cid < rid)
            cnt = cnt + jnp.sum(jnp.where(gt | tie, 1.0, 0.0),
                                axis=1, keepdims=True)
        o_ref[...] = cnt.astype(_i32)
    return pl.pallas_call(
        body,
        grid=(npc // BLK,),
        in_specs=[pl.BlockSpec((BLK, 1), lambda i: (i, 0)),
                  pl.BlockSpec((1, npc), lambda i: (0, 0))],
        out_specs=pl.BlockSpec((BLK, 1), lambda i: (i, 0)),
        out_shape=jax.ShapeDtypeStruct((npc, 1), _i32),
    )(score_col, score_row)


def kernel(x, edge_index, batch, W_down, b_down, p_pool, W_up, b_up):
    NP0 = _NPAD[_N]
    x_p = jnp.zeros((NP0, _D), _f32).at[:_N].set(x)
    b_p = jnp.zeros((NP0,), _i32).at[:_N].set(batch)
    ones0 = jnp.ones((NP0, 1), _f32)

    s0 = edge_index[0]
    d0 = edge_index[1]
    deff0, deg0 = _edge_init_kernel()(s0, d0)

    h, hs, dv = _prep(x_p, ones0, deg0[:NP0][:, None], deg0[NP0:][:, None],
                      W_down[0], NP0)
    agg = _agg_kernel(NP0)(hs, s0, deff0)
    xc = _epi(agg[0], agg[1], h, dv, b_down[0][None, :], _N, NP0, True)

    xs = [xc]
    saved = [(s0, deff0, deg0, _N, NP0)]
    perms = []
    b_cur = b_p
    s_arr, d_arr, deff = s0, d0, deff0
    n_cur, np_cur = _N, NP0
    for i in range(_DEPTH):
        kk = _KS[i]
        kp = _NPAD[kk]
        score = _score(xc, p_pool[i][:, None], np_cur)
        rank = _rank(score, score.reshape(1, np_cur), n_cur, np_cur)
        perm, new_s, new_d, new_deff, degp = _pool_a_kernel(
            n_cur, kk, np_cur, kp)(rank[:, 0], s_arr, d_arr, deff)
        xg, sg, bg = _pool_b_kernel(np_cur, kp)(
            perm, xc, score[:, 0], b_cur)
        h, hs, dv = _prep(xg, sg[:, None], degp[:kp][:, None], degp[kp:][:, None],
                          W_down[i + 1], kp)
        agg = _agg_kernel(kp)(hs, new_s, new_deff)
        xc = _epi(agg[0], agg[1], h, dv, b_down[i + 1][None, :], kk, kp, True)
        perms.append(perm)
        b_cur = bg
        s_arr, d_arr, deff = new_s, new_d, new_deff
        n_cur, np_cur = kk, kp
        if i < _DEPTH - 1:
            xs.append(xc)
            saved.append((new_s, new_deff, degp, kk, kp))

    latent_x = xc[:_KS[-1]]
    latent_edge = jnp.stack([s_arr, d_arr])
    latent_batch = b_cur[:_KS[-1]]

    for i in range(_DEPTH):
        j = _DEPTH - 1 - i
        s_j, deff_j, deg_j, n_j, np_j = saved[j]
        up = _unpool_kernel(np_cur, np_j)(xc, perms[j])
        h, hs, dv = _prep_up(xs[j], up[0], up[1],
                             deg_j[:np_j][:, None], deg_j[np_j:][:, None],
                             W_up[i], np_j)
        agg = _agg_kernel(np_j)(hs, s_j, deff_j)
        xc = _epi(agg[0], agg[1], h, dv, b_up[i][None, :],
                  n_j, np_j, i < _DEPTH - 1)
        n_cur, np_cur = n_j, np_j

    return xc[:_N], latent_x, latent_edge, latent_batch
